# baseline (speedup 1.0000x reference)
import functools

import jax
import jax.numpy as jnp
from jax import lax
from jax.experimental import pallas as pl
from jax.experimental.pallas import tpu as pltpu
from jax.experimental.pallas import tpu_sc as plsc

SIZE = 1000000
DIM = 32
B = 4096
L = 52
NLANES = 16

NC = 2
NS = 16
NW = NC * NS
BPW = B // NW
C = 8
CL = C * L
NCHUNK = BPW // C
NPAIR = NCHUNK // 2
JBASES = (0, 16, 32, 35)
GSLICES = ((0, 128), (128, 128), (256, 128), (384, 32))

_mesh = plsc.VectorSubcoreMesh(
    core_axis_name="c", subcore_axis_name="s", num_cores=NC, num_subcores=NS
)


RW = 512
RSTEPS = RW // NLANES
NB = 1953
RIN_B = 32 * RW * 4
ROUT_B = 128 * 128 * 4


@functools.partial(
    pl.kernel,
    out_type=jax.ShapeDtypeStruct((SIZE // 4, 128), jnp.float32),
    mesh=_mesh,
    scratch_types=[
        pltpu.VMEM((DIM, RW), jnp.float32),
        pltpu.VMEM((DIM, RW), jnp.float32),
        pltpu.VMEM((128, 128), jnp.float32),
        pltpu.VMEM((128, 128), jnp.float32),
        pltpu.SemaphoreType.DMA,
        pltpu.SemaphoreType.DMA,
        pltpu.SemaphoreType.DMA,
        pltpu.SemaphoreType.DMA,
    ],
    compiler_params=pltpu.CompilerParams(
        needs_layout_passes=False, use_tc_tiling_on_sc=True
    ),
)
def _sc_repack(tab_t, tail16, out_hbm, in_a, in_b, out_a, out_b, s_ia, s_ib, s_oa, s_ob):
    wid = lax.axis_index("s") * NC + lax.axis_index("c")
    s = (wid * NB) // NW
    e = ((wid + 1) * NB) // NW
    n = e - s
    npair = n // 2
    odd = n - 2 * npair

    iota = lax.iota(jnp.int32, NLANES)
    rdiv4 = lax.shift_right_logical(iota, 2)
    cpat = lax.shift_left(iota & 3, 5)

    def in_copy(t, in_ref, sem):
        return pltpu.make_async_copy(
            tab_t.at[:, pl.ds(t * RW, RW)], in_ref, sem
        )

    def out_copy(t, out_ref, sem):
        return pltpu.make_async_copy(
            out_ref, out_hbm.at[pl.ds(t * 128, 128)], sem
        )

    def start_in(t, in_ref, sem):
        in_copy(t, in_ref, sem).start()

    def start_out(t, out_ref, sem):
        out_copy(t, out_ref, sem).start()

    def repack_block(in_ref, out_ref):
        unroll = 16

        def body(u, carry):
            for v in range(unroll):
                c0 = u * unroll + v
                cvec = (c0 + iota) & (RW - 1)
                row = lax.shift_right_logical(cvec, 2)
                colbase = lax.shift_left(cvec & 3, 5)
                for d0 in (0, 16):
                    dvec = iota + d0
                    vals = plsc.load_gather(in_ref, [dvec, cvec])
                    plsc.store_scatter(out_ref, [row, colbase + dvec], vals)
            return carry

        lax.fori_loop(0, RW // unroll, body, 0)

    start_in(s, in_a, s_ia)

    def pair_body(i, carry):
        t_a = s + 2 * i
        t_b = t_a + 1
        start_in(t_b, in_b, s_ib)
        in_copy(t_a, in_a, s_ia).wait()

        @pl.when(i > 0)
        def _():
            out_copy(t_a - 2, out_a, s_oa).wait()

        repack_block(in_a, out_a)
        start_out(t_a, out_a, s_oa)

        @pl.when(t_b + 1 < e)
        def _():
            start_in(t_b + 1, in_a, s_ia)

        in_copy(t_b, in_b, s_ib).wait()

        @pl.when(i > 0)
        def _():
            out_copy(t_b - 2, out_b, s_ob).wait()

        repack_block(in_b, out_b)
        start_out(t_b, out_b, s_ob)
        return carry

    lax.fori_loop(0, npair, pair_body, 0)

    @pl.when(odd == 1)
    def _():
        in_copy(e - 1, in_a, s_ia).wait()
        out_copy(s + 2 * (npair - 1), out_a, s_oa).wait()
        repack_block(in_a, out_a)
        start_out(e - 1, out_a, s_oa)

    out_copy(e - 2 + odd, out_a, s_oa).wait()
    out_copy(e - 1 - odd, out_b, s_ob).wait()

    @pl.when(wid == NW - 1)
    def _():
        tbuf = in_a.at[pl.ds(0, 16), pl.ds(0, 128)]
        pltpu.sync_copy(tail16, tbuf)
        pltpu.sync_copy(tbuf, out_hbm.at[pl.ds(NB * 128, 16)])


@functools.partial(
    pl.kernel,
    out_type=jax.ShapeDtypeStruct((B, L - 1), jnp.float32),
    mesh=_mesh,
    scratch_types=[
        pltpu.VMEM((BPW * L,), jnp.int32),
        pltpu.VMEM((CL,), jnp.int32),
        pltpu.VMEM((CL,), jnp.int32),
        pltpu.VMEM((CL, 128), jnp.float32),
        pltpu.VMEM((CL, 128), jnp.float32),
        pltpu.VMEM((C, L - 1), jnp.float32),
        pltpu.SemaphoreType.DMA,
        pltpu.SemaphoreType.DMA,
    ],
    compiler_params=pltpu.CompilerParams(
        needs_layout_passes=False, use_tc_tiling_on_sc=True
    ),
)
def _sc_embed_dist(
    inputs_hbm, table_hbm, out_hbm,
    idx_all, q_a, q_b, rows_a, rows_b, out_v, sem_a, sem_b,
):
    wid = lax.axis_index("s") * NC + lax.axis_index("c")
    base = wid * BPW
    pltpu.sync_copy(inputs_hbm.at[pl.ds(base * L, BPW * L)], idx_all)

    iota = lax.iota(jnp.int32, NLANES)

    def qfill(k, qref):
        for t in range(CL // NLANES):
            v = idx_all[pl.ds(k * CL + t * NLANES, NLANES)]
            qref[pl.ds(t * NLANES, NLANES)] = lax.shift_right_logical(v, 2)

    def fire(qref, rows_ref, sem):
        for o, n in GSLICES:
            pltpu.make_async_copy(
                table_hbm.at[qref.at[pl.ds(o, n)]],
                rows_ref.at[pl.ds(o, n)],
                sem,
            ).start()

    def drain(qref, rows_ref, sem):
        for o, n in GSLICES:
            pltpu.make_async_copy(
                table_hbm.at[qref.at[pl.ds(o, n)]],
                rows_ref.at[pl.ds(o, n)],
                sem,
            ).wait()

    diagd = [(lax.iota(jnp.int32, NLANES) + d0) & 31 for d0 in range(DIM)]

    def compute(k, rows_ref, out_hbm_row):
        def row_body(r, carry):
            roff = r * L
            av = idx_all[pl.ds(k * CL + roff, NLANES)]
            cb0 = (av[0] & 3) * 32
            ridx, colb, accs = [], [], []
            for jb in JBASES:
                pos = roff + 1 + jb + iota
                idx_j = plsc.load_gather(idx_all, [k * CL + pos])
                ridx.append(pos)
                colb.append(lax.shift_left(idx_j & 3, 5))
                accs.append(jnp.zeros((NLANES,), jnp.float32))
            pos0 = jnp.full((NLANES,), roff, jnp.int32)
            for d0 in range(DIM):
                sv = plsc.load_gather(rows_ref, [pos0, cb0 + diagd[d0]])
                for g in range(len(JBASES)):
                    v = plsc.load_gather(rows_ref, [ridx[g], colb[g] + diagd[d0]])
                    diff = v - sv
                    accs[g] = accs[g] + diff * diff
            for g, jb in enumerate(JBASES):
                out_v[r, pl.ds(jb, NLANES)] = -accs[g]
            return carry

        lax.fori_loop(0, C, row_body, 0)
        pltpu.sync_copy(out_v, out_hbm.at[pl.ds(out_hbm_row, C)])

    qfill(0, q_a)
    fire(q_a, rows_a, sem_a)

    def pair_body(i, carry):
        k0 = 2 * i
        k1 = 2 * i + 1
        qfill(k1, q_b)
        fire(q_b, rows_b, sem_b)
        drain(q_a, rows_a, sem_a)
        compute(k0, rows_a, base + k0 * C)
        qfill(k1 + 1, q_a)
        fire(q_a, rows_a, sem_a)
        drain(q_b, rows_b, sem_b)
        compute(k1, rows_b, base + k1 * C)
        return carry

    lax.fori_loop(0, NPAIR - 1, pair_body, 0)

    qfill(NCHUNK - 1, q_b)
    fire(q_b, rows_b, sem_b)
    drain(q_a, rows_a, sem_a)
    compute(NCHUNK - 2, rows_a, base + (NCHUNK - 2) * C)
    drain(q_b, rows_b, sem_b)
    compute(NCHUNK - 1, rows_b, base + (NCHUNK - 1) * C)


def kernel(inputs, table):
    tail16 = jnp.reshape(table[NB * RW:], (16, 128))
    table_packed = _sc_repack(jnp.transpose(table), tail16)
    inputs_flat = jnp.reshape(inputs, (B * L,))
    return _sc_embed_dist(inputs_flat, table_packed)


# speedup vs baseline: 1.0046x; 1.0046x over previous
import functools

import jax
import jax.numpy as jnp
from jax import lax
from jax.experimental import pallas as pl
from jax.experimental.pallas import tpu as pltpu
from jax.experimental.pallas import tpu_sc as plsc

SIZE = 1000000
DIM = 32
B = 4096
L = 52
NLANES = 16

NC = 2
NS = 16
NW = NC * NS
BPW = B // NW
C = 8
CL = C * L
NCHUNK = BPW // C
NPAIR = NCHUNK // 2
JBASES = (0, 16, 32, 35)
GSLICES = ((0, 128), (128, 128), (256, 128), (384, 32))

_mesh = plsc.VectorSubcoreMesh(
    core_axis_name="c", subcore_axis_name="s", num_cores=NC, num_subcores=NS
)


RW = 512
NB = 1953
RIN_B = 32 * RW * 4
ROUT_B = 128 * 128 * 4


@functools.partial(
    pl.kernel,
    out_type=jax.ShapeDtypeStruct((SIZE // 4, 128), jnp.float32),
    mesh=_mesh,
    scratch_types=[
        pltpu.VMEM((DIM, RW), jnp.float32),
        pltpu.VMEM((DIM, RW), jnp.float32),
        pltpu.VMEM((128, 128), jnp.float32),
        pltpu.VMEM((128, 128), jnp.float32),
        pltpu.SemaphoreType.DMA,
        pltpu.SemaphoreType.DMA,
        pltpu.SemaphoreType.DMA,
        pltpu.SemaphoreType.DMA,
    ],
    compiler_params=pltpu.CompilerParams(
        needs_layout_passes=False, use_tc_tiling_on_sc=True
    ),
)
def _sc_repack(tab_t, tail16, out_hbm, in_a, in_b, out_a, out_b, s_ia, s_ib, s_oa, s_ob):
    wid = lax.axis_index("s") * NC + lax.axis_index("c")
    s = (wid * NB) // NW
    e = ((wid + 1) * NB) // NW
    n = e - s
    npair = n // 2
    odd = n - 2 * npair

    iota = lax.iota(jnp.int32, NLANES)

    def in_copy(t, in_ref, sem):
        return pltpu.make_async_copy(
            tab_t.at[:, pl.ds(t * RW, RW)], in_ref, sem
        )

    def out_copy(t, out_ref, sem):
        return pltpu.make_async_copy(
            out_ref, out_hbm.at[pl.ds(t * 128, 128)], sem
        )

    def start_in(t, in_ref, sem):
        in_copy(t, in_ref, sem).start()

    def start_out(t, out_ref, sem):
        out_copy(t, out_ref, sem).start()

    def repack_block(in_ref, out_ref):
        unroll = 16

        def body(u, carry):
            for v in range(unroll):
                c0 = u * unroll + v
                cvec = (c0 + iota) & (RW - 1)
                row = lax.shift_right_logical(cvec, 2)
                colbase = lax.shift_left(cvec & 3, 5)
                for d0 in (0, 16):
                    dvec = iota + d0
                    vals = plsc.load_gather(in_ref, [dvec, cvec])
                    plsc.store_scatter(out_ref, [row, colbase + dvec], vals)
            return carry

        lax.fori_loop(0, RW // unroll, body, 0)

    start_in(s, in_a, s_ia)

    def pair_body(i, carry):
        t_a = s + 2 * i
        t_b = t_a + 1
        start_in(t_b, in_b, s_ib)
        in_copy(t_a, in_a, s_ia).wait()

        @pl.when(i > 0)
        def _():
            out_copy(t_a - 2, out_a, s_oa).wait()

        repack_block(in_a, out_a)
        start_out(t_a, out_a, s_oa)

        @pl.when(t_b + 1 < e)
        def _():
            start_in(t_b + 1, in_a, s_ia)

        in_copy(t_b, in_b, s_ib).wait()

        @pl.when(i > 0)
        def _():
            out_copy(t_b - 2, out_b, s_ob).wait()

        repack_block(in_b, out_b)
        start_out(t_b, out_b, s_ob)
        return carry

    lax.fori_loop(0, npair, pair_body, 0)

    @pl.when(odd == 1)
    def _():
        in_copy(e - 1, in_a, s_ia).wait()
        out_copy(s + 2 * (npair - 1), out_a, s_oa).wait()
        repack_block(in_a, out_a)
        start_out(e - 1, out_a, s_oa)

    out_copy(e - 2 + odd, out_a, s_oa).wait()
    out_copy(e - 1 - odd, out_b, s_ob).wait()

    @pl.when(wid == NW - 1)
    def _():
        tbuf = in_a.at[pl.ds(0, 16), pl.ds(0, 128)]
        pltpu.sync_copy(tail16, tbuf)
        pltpu.sync_copy(tbuf, out_hbm.at[pl.ds(NB * 128, 16)])


@functools.partial(
    pl.kernel,
    out_type=jax.ShapeDtypeStruct((B, L - 1), jnp.float32),
    mesh=_mesh,
    scratch_types=[
        pltpu.VMEM((BPW * L,), jnp.int32),
        pltpu.VMEM((CL,), jnp.int32),
        pltpu.VMEM((CL,), jnp.int32),
        pltpu.VMEM((CL, 128), jnp.float32),
        pltpu.VMEM((CL, 128), jnp.float32),
        pltpu.VMEM((C, L - 1), jnp.float32),
        pltpu.SemaphoreType.DMA,
        pltpu.SemaphoreType.DMA,
    ],
    compiler_params=pltpu.CompilerParams(
        needs_layout_passes=False, use_tc_tiling_on_sc=True
    ),
)
def _sc_embed_dist(
    inputs_hbm, table_hbm, out_hbm,
    idx_all, q_a, q_b, rows_a, rows_b, out_v, sem_a, sem_b,
):
    wid = lax.axis_index("s") * NC + lax.axis_index("c")
    base = wid * BPW
    pltpu.sync_copy(inputs_hbm.at[pl.ds(base * L, BPW * L)], idx_all)

    iota = lax.iota(jnp.int32, NLANES)

    def qfill(k, qref):
        for t in range(CL // NLANES):
            v = idx_all[pl.ds(k * CL + t * NLANES, NLANES)]
            qref[pl.ds(t * NLANES, NLANES)] = lax.shift_right_logical(v, 2)

    def fire(qref, rows_ref, sem):
        for o, n in GSLICES:
            pltpu.make_async_copy(
                table_hbm.at[qref.at[pl.ds(o, n)]],
                rows_ref.at[pl.ds(o, n)],
                sem,
            ).start()

    def drain(qref, rows_ref, sem):
        for o, n in GSLICES:
            pltpu.make_async_copy(
                table_hbm.at[qref.at[pl.ds(o, n)]],
                rows_ref.at[pl.ds(o, n)],
                sem,
            ).wait()

    diagd = [(lax.iota(jnp.int32, NLANES) + d0) & 31 for d0 in range(DIM)]

    def compute(k, rows_ref, out_hbm_row):
        def row_body(r, carry):
            roff = r * L
            av = idx_all[pl.ds(k * CL + roff, NLANES)]
            cb0 = (av[0] & 3) * 32
            ridx, colb, accs = [], [], []
            for jb in JBASES:
                pos = roff + 1 + jb + iota
                idx_j = plsc.load_gather(idx_all, [k * CL + pos])
                ridx.append(pos)
                colb.append(lax.shift_left(idx_j & 3, 5))
                accs.append(jnp.zeros((NLANES,), jnp.float32))
            pos0 = jnp.full((NLANES,), roff, jnp.int32)
            for d0 in range(DIM):
                sv = plsc.load_gather(rows_ref, [pos0, cb0 + diagd[d0]])
                for g in range(len(JBASES)):
                    v = plsc.load_gather(rows_ref, [ridx[g], colb[g] + diagd[d0]])
                    diff = v - sv
                    accs[g] = accs[g] + diff * diff
            for g, jb in enumerate(JBASES):
                out_v[r, pl.ds(jb, NLANES)] = -accs[g]
            return carry

        lax.fori_loop(0, C, row_body, 0)
        pltpu.sync_copy(out_v, out_hbm.at[pl.ds(out_hbm_row, C)])

    qfill(0, q_a)
    fire(q_a, rows_a, sem_a)

    def pair_body(i, carry):
        k0 = 2 * i
        k1 = 2 * i + 1
        qfill(k1, q_b)
        fire(q_b, rows_b, sem_b)
        drain(q_a, rows_a, sem_a)
        compute(k0, rows_a, base + k0 * C)
        qfill(k1 + 1, q_a)
        fire(q_a, rows_a, sem_a)
        drain(q_b, rows_b, sem_b)
        compute(k1, rows_b, base + k1 * C)
        return carry

    lax.fori_loop(0, NPAIR - 1, pair_body, 0)

    qfill(NCHUNK - 1, q_b)
    fire(q_b, rows_b, sem_b)
    drain(q_a, rows_a, sem_a)
    compute(NCHUNK - 2, rows_a, base + (NCHUNK - 2) * C)
    drain(q_b, rows_b, sem_b)
    compute(NCHUNK - 1, rows_b, base + (NCHUNK - 1) * C)


def kernel(inputs, table):
    tail16 = jnp.reshape(table[NB * RW:], (16, 128))
    table_packed = _sc_repack(jnp.transpose(table), tail16)
    inputs_flat = jnp.reshape(inputs, (B * L,))
    return _sc_embed_dist(inputs_flat, table_packed)


# speedup vs baseline: 1.9141x; 1.9054x over previous
import functools

import jax
import jax.numpy as jnp
from jax import lax
from jax.experimental import pallas as pl
from jax.experimental.pallas import tpu as pltpu
from jax.experimental.pallas import tpu_sc as plsc

SIZE = 1000000
DIM = 32
B = 4096
L = 52
NLANES = 16

NC = 2
NS = 16
NW = NC * NS
BPW = B // NW
C = 8
CL = C * L
NCHUNK = BPW // C
NPAIR = NCHUNK // 2
JBASES = (0, 16, 32, 35)
GSLICES = ((0, 128), (128, 128), (256, 128), (384, 32))

_mesh = plsc.VectorSubcoreMesh(
    core_axis_name="c", subcore_axis_name="s", num_cores=NC, num_subcores=NS
)


RW = 512
NB = 1953
RIN_B = 32 * RW * 4
ROUT_B = 128 * 128 * 4


@functools.partial(
    pl.kernel,
    out_type=jax.ShapeDtypeStruct((SIZE // 4, 128), jnp.float32),
    mesh=_mesh,
    scratch_types=[
        pltpu.VMEM((DIM, RW), jnp.float32),
        pltpu.VMEM((DIM, RW), jnp.float32),
        pltpu.VMEM((128, 128), jnp.float32),
        pltpu.VMEM((128, 128), jnp.float32),
        pltpu.SemaphoreType.DMA,
        pltpu.SemaphoreType.DMA,
        pltpu.SemaphoreType.DMA,
        pltpu.SemaphoreType.DMA,
    ],
    compiler_params=pltpu.CompilerParams(
        needs_layout_passes=False, use_tc_tiling_on_sc=True
    ),
)
def _sc_repack(tab_t, tail16, out_hbm, in_a, in_b, out_a, out_b, s_ia, s_ib, s_oa, s_ob):
    wid = lax.axis_index("s") * NC + lax.axis_index("c")
    s = (wid * NB) // NW
    e = ((wid + 1) * NB) // NW
    n = e - s
    npair = n // 2
    odd = n - 2 * npair

    iota = lax.iota(jnp.int32, NLANES)

    def in_copy(t, in_ref, sem):
        return pltpu.make_async_copy(
            tab_t.at[:, pl.ds(t * RW, RW)], in_ref, sem
        )

    def out_copy(t, out_ref, sem):
        return pltpu.make_async_copy(
            out_ref, out_hbm.at[pl.ds(t * 128, 128)], sem
        )

    def start_in(t, in_ref, sem):
        in_copy(t, in_ref, sem).start()

    def start_out(t, out_ref, sem):
        out_copy(t, out_ref, sem).start()

    def repack_block(in_ref, out_ref):
        unroll = 8

        def body(u, carry):
            recs = []
            for v in range(unroll):
                c0 = u * unroll + v
                cvec = (c0 + iota) & (RW - 1)
                row = lax.shift_right_logical(cvec, 2)
                colbase = lax.shift_left(cvec & 3, 5)
                for d0 in (0, 16):
                    dvec = iota + d0
                    vals = plsc.load_gather(in_ref, [dvec, cvec])
                    recs.append((row, colbase + dvec, vals))
            for row, col, vals in recs:
                plsc.store_scatter(out_ref, [row, col], vals)
            return carry

        lax.fori_loop(0, RW // unroll, body, 0)

    start_in(s, in_a, s_ia)

    def pair_body(i, carry):
        t_a = s + 2 * i
        t_b = t_a + 1
        start_in(t_b, in_b, s_ib)
        in_copy(t_a, in_a, s_ia).wait()

        @pl.when(i > 0)
        def _():
            out_copy(t_a - 2, out_a, s_oa).wait()

        repack_block(in_a, out_a)
        start_out(t_a, out_a, s_oa)

        @pl.when(t_b + 1 < e)
        def _():
            start_in(t_b + 1, in_a, s_ia)

        in_copy(t_b, in_b, s_ib).wait()

        @pl.when(i > 0)
        def _():
            out_copy(t_b - 2, out_b, s_ob).wait()

        repack_block(in_b, out_b)
        start_out(t_b, out_b, s_ob)
        return carry

    lax.fori_loop(0, npair, pair_body, 0)

    @pl.when(odd == 1)
    def _():
        in_copy(e - 1, in_a, s_ia).wait()
        out_copy(s + 2 * (npair - 1), out_a, s_oa).wait()
        repack_block(in_a, out_a)
        start_out(e - 1, out_a, s_oa)

    out_copy(e - 2 + odd, out_a, s_oa).wait()
    out_copy(e - 1 - odd, out_b, s_ob).wait()

    @pl.when(wid == NW - 1)
    def _():
        tbuf = in_a.at[pl.ds(0, 16), pl.ds(0, 128)]
        pltpu.sync_copy(tail16, tbuf)
        pltpu.sync_copy(tbuf, out_hbm.at[pl.ds(NB * 128, 16)])


@functools.partial(
    pl.kernel,
    out_type=jax.ShapeDtypeStruct((B, L - 1), jnp.float32),
    mesh=_mesh,
    scratch_types=[
        pltpu.VMEM((BPW * L,), jnp.int32),
        pltpu.VMEM((CL,), jnp.int32),
        pltpu.VMEM((CL,), jnp.int32),
        pltpu.VMEM((CL, 128), jnp.float32),
        pltpu.VMEM((CL, 128), jnp.float32),
        pltpu.VMEM((C, L - 1), jnp.float32),
        pltpu.SemaphoreType.DMA,
        pltpu.SemaphoreType.DMA,
    ],
    compiler_params=pltpu.CompilerParams(
        needs_layout_passes=False, use_tc_tiling_on_sc=True
    ),
)
def _sc_embed_dist(
    inputs_hbm, table_hbm, out_hbm,
    idx_all, q_a, q_b, rows_a, rows_b, out_v, sem_a, sem_b,
):
    wid = lax.axis_index("s") * NC + lax.axis_index("c")
    base = wid * BPW
    pltpu.sync_copy(inputs_hbm.at[pl.ds(base * L, BPW * L)], idx_all)

    iota = lax.iota(jnp.int32, NLANES)

    def qfill(k, qref):
        for t in range(CL // NLANES):
            v = idx_all[pl.ds(k * CL + t * NLANES, NLANES)]
            qref[pl.ds(t * NLANES, NLANES)] = lax.shift_right_logical(v, 2)

    def fire(qref, rows_ref, sem):
        for o, n in GSLICES:
            pltpu.make_async_copy(
                table_hbm.at[qref.at[pl.ds(o, n)]],
                rows_ref.at[pl.ds(o, n)],
                sem,
            ).start()

    def drain(qref, rows_ref, sem):
        for o, n in GSLICES:
            pltpu.make_async_copy(
                table_hbm.at[qref.at[pl.ds(o, n)]],
                rows_ref.at[pl.ds(o, n)],
                sem,
            ).wait()

    diagd = [(lax.iota(jnp.int32, NLANES) + d0) & 31 for d0 in range(DIM)]

    def compute(k, rows_ref, out_hbm_row):
        def row_body(r, carry):
            roff = r * L
            av = idx_all[pl.ds(k * CL + roff, NLANES)]
            cb0 = (av[0] & 3) * 32
            ridx, colb, accs = [], [], []
            for jb in JBASES:
                pos = roff + 1 + jb + iota
                idx_j = plsc.load_gather(idx_all, [k * CL + pos])
                ridx.append(pos)
                colb.append(lax.shift_left(idx_j & 3, 5))
                accs.append(jnp.zeros((NLANES,), jnp.float32))
            pos0 = jnp.full((NLANES,), roff, jnp.int32)
            for d0 in range(DIM):
                sv = plsc.load_gather(rows_ref, [pos0, cb0 + diagd[d0]])
                for g in range(len(JBASES)):
                    v = plsc.load_gather(rows_ref, [ridx[g], colb[g] + diagd[d0]])
                    diff = v - sv
                    accs[g] = accs[g] + diff * diff
            for g, jb in enumerate(JBASES):
                out_v[r, pl.ds(jb, NLANES)] = -accs[g]
            return carry

        lax.fori_loop(0, C, row_body, 0)
        pltpu.sync_copy(out_v, out_hbm.at[pl.ds(out_hbm_row, C)])

    qfill(0, q_a)
    fire(q_a, rows_a, sem_a)

    def pair_body(i, carry):
        k0 = 2 * i
        k1 = 2 * i + 1
        qfill(k1, q_b)
        fire(q_b, rows_b, sem_b)
        drain(q_a, rows_a, sem_a)
        compute(k0, rows_a, base + k0 * C)
        qfill(k1 + 1, q_a)
        fire(q_a, rows_a, sem_a)
        drain(q_b, rows_b, sem_b)
        compute(k1, rows_b, base + k1 * C)
        return carry

    lax.fori_loop(0, NPAIR - 1, pair_body, 0)

    qfill(NCHUNK - 1, q_b)
    fire(q_b, rows_b, sem_b)
    drain(q_a, rows_a, sem_a)
    compute(NCHUNK - 2, rows_a, base + (NCHUNK - 2) * C)
    drain(q_b, rows_b, sem_b)
    compute(NCHUNK - 1, rows_b, base + (NCHUNK - 1) * C)


def kernel(inputs, table):
    tail16 = jnp.reshape(table[NB * RW:], (16, 128))
    table_packed = _sc_repack(jnp.transpose(table), tail16)
    inputs_flat = jnp.reshape(inputs, (B * L,))
    return _sc_embed_dist(inputs_flat, table_packed)


# speedup vs baseline: 1.9173x; 1.0017x over previous
import functools

import jax
import jax.numpy as jnp
from jax import lax
from jax.experimental import pallas as pl
from jax.experimental.pallas import tpu as pltpu
from jax.experimental.pallas import tpu_sc as plsc

SIZE = 1000000
DIM = 32
B = 4096
L = 52
NLANES = 16

NC = 2
NS = 16
NW = NC * NS
BPW = B // NW
C = 8
CL = C * L
NCHUNK = BPW // C
NPAIR = NCHUNK // 2
JBASES = (0, 16, 32, 35)
GSLICES = ((0, 128), (128, 128), (256, 128), (384, 32))

_mesh = plsc.VectorSubcoreMesh(
    core_axis_name="c", subcore_axis_name="s", num_cores=NC, num_subcores=NS
)


RW = 512
NB = 1953
RIN_B = 32 * RW * 4
ROUT_B = 128 * 128 * 4


@functools.partial(
    pl.kernel,
    out_type=jax.ShapeDtypeStruct((SIZE // 4, 128), jnp.float32),
    mesh=_mesh,
    scratch_types=[
        pltpu.VMEM((DIM, RW), jnp.float32),
        pltpu.VMEM((DIM, RW), jnp.float32),
        pltpu.VMEM((128, 128), jnp.float32),
        pltpu.VMEM((128, 128), jnp.float32),
        pltpu.SemaphoreType.DMA,
        pltpu.SemaphoreType.DMA,
        pltpu.SemaphoreType.DMA,
        pltpu.SemaphoreType.DMA,
    ],
    compiler_params=pltpu.CompilerParams(
        needs_layout_passes=False, use_tc_tiling_on_sc=True
    ),
)
def _sc_repack(tab_t, tail16, out_hbm, in_a, in_b, out_a, out_b, s_ia, s_ib, s_oa, s_ob):
    wid = lax.axis_index("s") * NC + lax.axis_index("c")
    s = (wid * NB) // NW
    e = ((wid + 1) * NB) // NW
    n = e - s
    npair = n // 2
    odd = n - 2 * npair

    iota = lax.iota(jnp.int32, NLANES)

    def in_copy(t, in_ref, sem):
        return pltpu.make_async_copy(
            tab_t.at[:, pl.ds(t * RW, RW)], in_ref, sem
        )

    def out_copy(t, out_ref, sem):
        return pltpu.make_async_copy(
            out_ref, out_hbm.at[pl.ds(t * 128, 128)], sem
        )

    def start_in(t, in_ref, sem):
        in_copy(t, in_ref, sem).start()

    def start_out(t, out_ref, sem):
        out_copy(t, out_ref, sem).start()

    def repack_block(in_ref, out_ref):
        unroll = 8

        def body(u, carry):
            recs = []
            for v in range(unroll):
                c0 = u * unroll + v
                cvec = (c0 + iota) & (RW - 1)
                row = lax.shift_right_logical(cvec, 2)
                colbase = lax.shift_left(cvec & 3, 5)
                for d0 in (0, 16):
                    dvec = iota + d0
                    vals = plsc.load_gather(in_ref, [dvec, cvec])
                    recs.append((row, colbase + dvec, vals))
            for row, col, vals in recs:
                plsc.store_scatter(out_ref, [row, col], vals)
            return carry

        lax.fori_loop(0, RW // unroll, body, 0)

    start_in(s, in_a, s_ia)

    def pair_body(i, carry):
        t_a = s + 2 * i
        t_b = t_a + 1
        start_in(t_b, in_b, s_ib)
        in_copy(t_a, in_a, s_ia).wait()

        @pl.when(i > 0)
        def _():
            out_copy(t_a - 2, out_a, s_oa).wait()

        repack_block(in_a, out_a)
        start_out(t_a, out_a, s_oa)

        @pl.when(t_b + 1 < e)
        def _():
            start_in(t_b + 1, in_a, s_ia)

        in_copy(t_b, in_b, s_ib).wait()

        @pl.when(i > 0)
        def _():
            out_copy(t_b - 2, out_b, s_ob).wait()

        repack_block(in_b, out_b)
        start_out(t_b, out_b, s_ob)
        return carry

    lax.fori_loop(0, npair, pair_body, 0)

    @pl.when(odd == 1)
    def _():
        in_copy(e - 1, in_a, s_ia).wait()
        out_copy(s + 2 * (npair - 1), out_a, s_oa).wait()
        repack_block(in_a, out_a)
        start_out(e - 1, out_a, s_oa)

    out_copy(e - 2 + odd, out_a, s_oa).wait()
    out_copy(e - 1 - odd, out_b, s_ob).wait()

    @pl.when(wid == NW - 1)
    def _():
        tbuf = in_a.at[pl.ds(0, 16), pl.ds(0, 128)]
        pltpu.sync_copy(tail16, tbuf)
        pltpu.sync_copy(tbuf, out_hbm.at[pl.ds(NB * 128, 16)])


@functools.partial(
    pl.kernel,
    out_type=jax.ShapeDtypeStruct((B, L - 1), jnp.float32),
    mesh=_mesh,
    scratch_types=[
        pltpu.VMEM((BPW * L,), jnp.int32),
        pltpu.VMEM((CL,), jnp.int32),
        pltpu.VMEM((CL,), jnp.int32),
        pltpu.VMEM((CL, 128), jnp.float32),
        pltpu.VMEM((CL, 128), jnp.float32),
        pltpu.VMEM((C, L - 1), jnp.float32),
        pltpu.SemaphoreType.DMA,
        pltpu.SemaphoreType.DMA,
    ],
    compiler_params=pltpu.CompilerParams(
        needs_layout_passes=False, use_tc_tiling_on_sc=True
    ),
)
def _sc_embed_dist(
    inputs_hbm, table_hbm, out_hbm,
    idx_all, q_a, q_b, rows_a, rows_b, out_v, sem_a, sem_b,
):
    wid = lax.axis_index("s") * NC + lax.axis_index("c")
    base = wid * BPW
    pltpu.sync_copy(inputs_hbm.at[pl.ds(base * L, BPW * L)], idx_all)

    iota = lax.iota(jnp.int32, NLANES)

    def qfill(k, qref):
        for t in range(CL // NLANES):
            v = idx_all[pl.ds(k * CL + t * NLANES, NLANES)]
            qref[pl.ds(t * NLANES, NLANES)] = lax.shift_right_logical(v, 2)

    def fire(qref, rows_ref, sem):
        for o, n in GSLICES:
            pltpu.make_async_copy(
                table_hbm.at[qref.at[pl.ds(o, n)]],
                rows_ref.at[pl.ds(o, n)],
                sem,
            ).start()

    def drain(qref, rows_ref, sem):
        for o, n in GSLICES:
            pltpu.make_async_copy(
                table_hbm.at[qref.at[pl.ds(o, n)]],
                rows_ref.at[pl.ds(o, n)],
                sem,
            ).wait()

    diagd = [(lax.iota(jnp.int32, NLANES) + d0) & 31 for d0 in range(DIM)]

    def compute(k, rows_ref, out_hbm_row):
        def row_body(r, carry):
            roff = r * L
            av = idx_all[pl.ds(k * CL + roff, NLANES)]
            cb0 = (av[0] & 3) * 32
            ridx, colb, accs = [], [], []
            for jb in JBASES:
                pos = roff + 1 + jb + iota
                idx_j = plsc.load_gather(idx_all, [k * CL + pos])
                ridx.append(pos)
                colb.append(lax.shift_left(idx_j & 3, 5))
                accs.append(jnp.zeros((NLANES,), jnp.float32))
            pos0 = jnp.full((NLANES,), roff, jnp.int32)
            for d0 in range(0, DIM, 2):
                recs = []
                for dd in (d0, d0 + 1):
                    sv = plsc.load_gather(rows_ref, [pos0, cb0 + diagd[dd]])
                    vs = [
                        plsc.load_gather(rows_ref, [ridx[g], colb[g] + diagd[dd]])
                        for g in range(len(JBASES))
                    ]
                    recs.append((sv, vs))
                for sv, vs in recs:
                    for g in range(len(JBASES)):
                        diff = vs[g] - sv
                        accs[g] = accs[g] + diff * diff
            for g, jb in enumerate(JBASES):
                out_v[r, pl.ds(jb, NLANES)] = -accs[g]
            return carry

        lax.fori_loop(0, C, row_body, 0)
        pltpu.sync_copy(out_v, out_hbm.at[pl.ds(out_hbm_row, C)])

    qfill(0, q_a)
    fire(q_a, rows_a, sem_a)

    def pair_body(i, carry):
        k0 = 2 * i
        k1 = 2 * i + 1
        qfill(k1, q_b)
        fire(q_b, rows_b, sem_b)
        drain(q_a, rows_a, sem_a)
        compute(k0, rows_a, base + k0 * C)
        qfill(k1 + 1, q_a)
        fire(q_a, rows_a, sem_a)
        drain(q_b, rows_b, sem_b)
        compute(k1, rows_b, base + k1 * C)
        return carry

    lax.fori_loop(0, NPAIR - 1, pair_body, 0)

    qfill(NCHUNK - 1, q_b)
    fire(q_b, rows_b, sem_b)
    drain(q_a, rows_a, sem_a)
    compute(NCHUNK - 2, rows_a, base + (NCHUNK - 2) * C)
    drain(q_b, rows_b, sem_b)
    compute(NCHUNK - 1, rows_b, base + (NCHUNK - 1) * C)


def kernel(inputs, table):
    tail16 = jnp.reshape(table[NB * RW:], (16, 128))
    table_packed = _sc_repack(jnp.transpose(table), tail16)
    inputs_flat = jnp.reshape(inputs, (B * L,))
    return _sc_embed_dist(inputs_flat, table_packed)
